# baseline (speedup 1.0000x reference)
import jax
import jax.numpy as jnp
from jax.experimental import pallas as pl
from jax.experimental.pallas import tpu as pltpu
from jax.experimental.pallas import tpu_sc as plsc
import functools

B = 16384
A = 6
AP = 8
H = 256
D = 64
K = 1024
BETA = 0.25
RECONS_W = 1.0

BBLK = 2048
NBLK = B // BBLK


def _enc_kernel(a_ref, w1_ref, b1_ref, w2_ref, b2_ref, w3_ref, b3_ref, e_ref):
    h = jnp.maximum(jnp.dot(a_ref[...], w1_ref[...],
                            preferred_element_type=jnp.float32) + b1_ref[...], 0.0)
    h = jnp.maximum(jnp.dot(h, w2_ref[...],
                            preferred_element_type=jnp.float32) + b2_ref[...], 0.0)
    e_ref[...] = jnp.dot(h, w3_ref[...],
                         preferred_element_type=jnp.float32) + b3_ref[...]


def _vq_kernel(e_ref, en_ref, cbt_ref, idx_ref):
    cbt = cbt_ref[...]
    cn = jnp.sum(cbt * cbt, axis=0, keepdims=True)
    d2 = (en_ref[...] - 2.0 * jnp.dot(e_ref[...], cbt,
                                      preferred_element_type=jnp.float32)) + cn
    m2 = jnp.min(d2, axis=1, keepdims=True)
    y = jnp.maximum(m2, 0.0)
    s = jnp.sqrt(y)
    yb = jax.lax.bitcast_convert_type(y, jnp.int32)
    kinc = jax.lax.broadcasted_iota(jnp.int32, (1, 32), 1)
    cand = jax.lax.bitcast_convert_type(yb + kinc, jnp.float32)
    ok = jnp.sqrt(cand) <= s
    t = jnp.max(jnp.where(ok, cand, y), axis=1, keepdims=True)
    lanes = jax.lax.broadcasted_iota(jnp.int32, (1, K), 1)
    idx = jnp.min(jnp.where(d2 <= t, lanes, jnp.int32(K)), axis=1).astype(jnp.int32)
    idx_ref[...] = idx[None, None, :]


_SC_INFO = plsc.get_sparse_core_info()
_NW = _SC_INFO.num_cores * _SC_INFO.num_subcores
_BPW = B // _NW


def _sc_gather(codebook_p, idx):
    mesh = plsc.VectorSubcoreMesh(core_axis_name="c", subcore_axis_name="s")

    @functools.partial(
        pl.kernel, mesh=mesh,
        out_type=jax.ShapeDtypeStruct((B, 128), jnp.float32),
        scratch_types=[
            pltpu.VMEM((_BPW,), jnp.int32),
            pltpu.VMEM((_BPW, 128), jnp.float32),
            pltpu.SemaphoreType.DMA,
        ],
    )
    def k(table_hbm, idx_hbm, out_hbm, idx_v, rows_v, sem):
        wid = jax.lax.axis_index("s") * _SC_INFO.num_cores + jax.lax.axis_index("c")
        base = wid * _BPW
        pltpu.sync_copy(idx_hbm.at[pl.ds(base, _BPW)], idx_v)
        pltpu.async_copy(table_hbm.at[idx_v], rows_v, sem).wait()
        pltpu.sync_copy(rows_v, out_hbm.at[pl.ds(base, _BPW)])

    return k(codebook_p, idx)


def _dec_kernel(a_ref, e_ref, q_ref, wd1_ref, bd1_ref, wd2_ref, bd2_ref,
                wd3_ref, bd3_ref, qst_ref, acc_ref, scal_ref):
    i = pl.program_id(0)
    e = e_ref[...]
    q = q_ref[:, :D]
    qst = e + (q - e)
    qst_ref[...] = qst
    hd = jnp.maximum(jnp.dot(qst, wd1_ref[...],
                             preferred_element_type=jnp.float32) + bd1_ref[...], 0.0)
    hd = jnp.maximum(jnp.dot(hd, wd2_ref[...],
                             preferred_element_type=jnp.float32) + bd2_ref[...], 0.0)
    r = jnp.tanh(jnp.dot(hd, wd3_ref[...],
                         preferred_element_type=jnp.float32) + bd3_ref[...])
    dq = q - e
    dr = r - a_ref[...]
    lane = jax.lax.broadcasted_iota(jnp.int32, (1, 128), 1)
    vals = jnp.where(lane == 0, jnp.sum(dq * dq),
                     jnp.where(lane == 1, jnp.sum(dr * dr), 0.0))

    @pl.when(i == 0)
    def _init():
        acc_ref[...] = vals

    @pl.when(i > 0)
    def _accum():
        acc_ref[...] = acc_ref[...] + vals

    @pl.when(i == NBLK - 1)
    def _finalize():
        accv = acc_ref[...]
        s_vq = jnp.sum(jnp.where(lane == 0, accv, 0.0))
        s_rec = jnp.sum(jnp.where(lane == 1, accv, 0.0))
        vq_mse = s_vq / (B * D)
        vq_loss = vq_mse * BETA + vq_mse
        recons_loss = s_rec / (B * A)
        total = RECONS_W * recons_loss + vq_loss
        scal_ref[...] = jnp.where(
            lane == 0, total,
            jnp.where(lane == 1, recons_loss,
                      jnp.where(lane == 2, vq_loss, vq_mse)))


def kernel(action, W1, b1, W2, b2, W3, b3, codebook,
           Wd1, bd1, Wd2, bd2, Wd3, bd3):
    f32 = jnp.float32
    a_p = jnp.pad(action, ((0, 0), (0, AP - A)))
    w1_p = jnp.pad(W1, ((0, AP - A), (0, 0)))
    wd3_p = jnp.pad(Wd3, ((0, 0), (0, AP - A)))
    bd3_p = jnp.pad(bd3, (0, AP - A)).reshape(1, AP)
    cbt = codebook.T

    rep = lambda i: (0, 0)
    e = pl.pallas_call(
        _enc_kernel,
        grid=(NBLK,),
        in_specs=[
            pl.BlockSpec((BBLK, AP), lambda i: (i, 0)),
            pl.BlockSpec((AP, H), rep),
            pl.BlockSpec((1, H), rep),
            pl.BlockSpec((H, H), rep),
            pl.BlockSpec((1, H), rep),
            pl.BlockSpec((H, D), rep),
            pl.BlockSpec((1, D), rep),
        ],
        out_specs=pl.BlockSpec((BBLK, D), lambda i: (i, 0)),
        out_shape=jax.ShapeDtypeStruct((B, D), f32),
        compiler_params=pltpu.CompilerParams(
            dimension_semantics=("arbitrary",)),
    )(a_p, w1_p, b1.reshape(1, H), W2, b2.reshape(1, H), W3, b3.reshape(1, D))

    en = jnp.sum(e ** 2, axis=1, keepdims=True)

    idx_out = pl.pallas_call(
        _vq_kernel,
        grid=(NBLK,),
        in_specs=[
            pl.BlockSpec((BBLK, D), lambda i: (i, 0)),
            pl.BlockSpec((BBLK, 1), lambda i: (i, 0)),
            pl.BlockSpec((D, K), rep),
        ],
        out_specs=pl.BlockSpec((1, 1, BBLK), lambda i: (i, 0, 0)),
        out_shape=jax.ShapeDtypeStruct((NBLK, 1, B // NBLK), jnp.int32),
        compiler_params=pltpu.CompilerParams(
            dimension_semantics=("arbitrary",)),
    )(e, en, cbt)
    idx_flat = idx_out.reshape(B)

    cb_p = jnp.pad(codebook, ((0, 0), (0, 128 - D)))
    q_pad = _sc_gather(cb_p, idx_flat)

    qst_out, acc, scal = pl.pallas_call(
        _dec_kernel,
        grid=(NBLK,),
        in_specs=[
            pl.BlockSpec((BBLK, AP), lambda i: (i, 0)),
            pl.BlockSpec((BBLK, D), lambda i: (i, 0)),
            pl.BlockSpec((BBLK, 128), lambda i: (i, 0)),
            pl.BlockSpec((D, H), rep),
            pl.BlockSpec((1, H), rep),
            pl.BlockSpec((H, H), rep),
            pl.BlockSpec((1, H), rep),
            pl.BlockSpec((H, AP), rep),
            pl.BlockSpec((1, AP), rep),
        ],
        out_specs=[
            pl.BlockSpec((BBLK, D), lambda i: (i, 0)),
            pl.BlockSpec((1, 128), rep),
            pl.BlockSpec((1, 128), rep),
        ],
        out_shape=[
            jax.ShapeDtypeStruct((B, D), f32),
            jax.ShapeDtypeStruct((1, 128), f32),
            jax.ShapeDtypeStruct((1, 128), f32),
        ],
        compiler_params=pltpu.CompilerParams(
            dimension_semantics=("arbitrary",)),
    )(a_p, e, q_pad, Wd1, bd1.reshape(1, H), Wd2, bd2.reshape(1, H), wd3_p, bd3_p)

    quantized_index = idx_flat
    quantized_embedding = qst_out
    total = scal[0, 0]
    recons_loss = scal[0, 1]
    vq_loss = scal[0, 2]
    embedding_loss = scal[0, 3]
    commitment_loss = scal[0, 3]
    return (total, recons_loss, vq_loss, embedding_loss, commitment_loss,
            quantized_index, quantized_embedding)
